# baseline (speedup 1.0000x reference)
import functools

import jax
import jax.numpy as jnp
from jax import lax
from jax.experimental import pallas as pl
from jax.experimental.pallas import tpu as pltpu
from jax.experimental.pallas import tpu_sc as plsc

NUM_BUCKETS = 1000000
EMBED_DIM = 32
BATCH = 4
SEQ_LEN = 8192
FLAT = BATCH * SEQ_LEN

A_HI = 761000
A_LO = 435761
B_HI = 503000
B_LO = 40503

_INFO = plsc.get_sparse_core_info()
NC = _INFO.num_cores
NS = _INFO.num_subcores
L = _INFO.num_lanes
CHUNK = FLAT // NS
STEPS = CHUNK // L
HALF = EMBED_DIM // NC

PW = 32768
SWEEP_END = 999936
NPIECE = 31
LAST_W = SWEEP_END - (NPIECE - 1) * PW
TAIL = NUM_BUCKETS - SWEEP_END


def _sc_body(ids_hbm, tab_hbm, tail_hbm, out_hbm, ids_v, idx_v, wb_v, wp_v,
             stage_v, chunk_v, cg_refs, gidx_v, tail_v, sh_v, sem, csem,
             rsem):
    c = lax.axis_index("c")
    s = lax.axis_index("s")
    base = s * jnp.int32(CHUNK)
    row = base // jnp.int32(SEQ_LEN)
    toff = base % jnp.int32(SEQ_LEN)
    fbase = c * jnp.int32(HALF)

    zeros = jnp.zeros((L,), jnp.int32)
    lane = lax.iota(jnp.int32, L)

    plsc.store_scatter(ids_v, [lane], zeros)
    pltpu.sync_copy(ids_hbm.at[pl.ds(base, CHUNK)], ids_v.at[pl.ds(8, CHUNK)])

    @pl.when(s % jnp.int32(NS // BATCH) != 0)
    def _():
        pltpu.sync_copy(ids_hbm.at[pl.ds(base - 8, 8)], ids_v.at[pl.ds(0, 8)])

    pltpu.sync_copy(tail_hbm, tail_v)

    a_hi = jnp.int32(A_HI)
    a_lo = jnp.int32(A_LO)
    b_hi = jnp.int32(B_HI)
    b_lo = jnp.int32(B_LO)
    thousand = jnp.int32(1000)
    nbuckets = jnp.int32(NUM_BUCKETS)

    def hash_step(_, off):
        cur = plsc.load_gather(ids_v, [lane + (off + jnp.int32(8))])
        prev = plsc.load_gather(ids_v, [lane + (off + jnp.int32(7))])
        p1 = prev // thousand
        p0 = prev - p1 * thousand
        c1 = cur // thousand
        c0 = cur - c1 * thousand
        h = (a_hi * p1 + a_lo * p0 + b_hi * c1 + b_lo * c0) % nbuckets
        plsc.store_scatter(idx_v, [lane + off], h)
        return off + jnp.int32(L)

    lax.fori_loop(0, STEPS, hash_step, jnp.int32(0), unroll=4)

    def scan_range(plo, phi, ref0):
        vlo = jnp.full((L,), plo, jnp.int32)
        vhi = jnp.full((L,), phi, jnp.int32)
        vref = jnp.full((L,), ref0, jnp.int32)

        def scan_step(_, carry):
            off, cnt = carry
            h = plsc.load_gather(idx_v, [lane + off])
            mask = jnp.logical_and(h >= vlo, h < vhi)
            plsc.store_compressed(wb_v.at[pl.ds(cnt, L)], h - vref, mask=mask)
            plsc.store_compressed(wp_v.at[pl.ds(cnt, L)], lane + off, mask=mask)
            nmask = plsc.all_reduce_population_count(mask)
            cnt = cnt + jnp.max(nmask)
            return off + jnp.int32(L), cnt

        _, m = lax.fori_loop(0, STEPS, scan_step,
                             (jnp.int32(0), jnp.int32(0)), unroll=4)
        return m

    def extract_from_piece(m):
        def sg_step(_, sg0):
            bvs = []
            wps = []
            vmasks = []
            for k in range(8):
                off = sg0 + jnp.int32(k * L)
                vmask = lane < (m - off)
                bv = plsc.load_gather(wb_v, [lane + off])
                bv = jnp.where(vmask, bv, 0)
                bvs.append(bv)
                wps.append(plsc.load_gather(wp_v, [lane + off]))
                vmasks.append(vmask)
            for d in range(HALF):
                dof = jnp.int32(d * PW)
                for k in range(8):
                    plsc.store_scatter(
                        gidx_v, [lane + jnp.int32(d * 128 + k * L)],
                        bvs[k] + dof)
            copies = []
            for d in range(HALF):
                copies.append(pltpu.async_copy(
                    sh_v.at[gidx_v.at[pl.ds(d * 128, 128)]],
                    cg_refs[d],
                    sem))
            for cp in copies:
                cp.wait()
            for k in range(8):
                for d in range(HALF):
                    vals = cg_refs[d][pl.ds(k * L, L)]
                    dv = jnp.full((L,), d, jnp.int32)
                    plsc.store_scatter(stage_v, [dv, wps[k]], vals,
                                       mask=vmasks[k])
            return sg0 + jnp.int32(128)

        nsg = (m + jnp.int32(127)) // jnp.int32(128)
        lax.fori_loop(0, nsg, sg_step, jnp.int32(0))

    def chunk_src(pc):
        plo = pc * jnp.int32(PW)
        lo_load = jnp.minimum(plo, jnp.int32(SWEEP_END - PW))
        lo_load = pl.multiple_of(lo_load, 128)
        blk = s % jnp.int32(2)
        part = s // jnp.int32(2)
        col0 = pl.multiple_of(lo_load + part * jnp.int32(PW // 8), 128)
        return tab_hbm.at[pl.ds(fbase + blk * 8, 8), pl.ds(col0, PW // 8)]

    pltpu.async_copy(chunk_src(jnp.int32(0)), chunk_v, csem)

    def piece_body(_, pc):
        plo = pc * jnp.int32(PW)
        phi = jnp.minimum(plo + jnp.int32(PW), jnp.int32(SWEEP_END))
        lo_load = jnp.minimum(plo, jnp.int32(SWEEP_END - PW))
        blk = s % jnp.int32(2)
        part = s // jnp.int32(2)
        pltpu.make_async_copy(chunk_src(pc), chunk_v, csem).wait()
        rows = []
        for d in range(8):
            dglob = blk * 8 + jnp.int32(d)
            rows.append(pltpu.async_copy(
                chunk_v.at[jnp.int32(d)],
                sh_v.at[pl.ds(dglob * jnp.int32(PW)
                              + part * jnp.int32(PW // 8), PW // 8)],
                rsem))
        for r in rows:
            r.wait()

        @pl.when(pc + 1 < jnp.int32(NPIECE))
        def _():
            pltpu.async_copy(chunk_src(pc + 1), chunk_v, csem)

        plsc.subcore_barrier()
        m = scan_range(plo, phi, lo_load)
        extract_from_piece(m)
        plsc.subcore_barrier()
        return pc + jnp.int32(1)

    lax.fori_loop(0, NPIECE, piece_body, jnp.int32(0))

    m = scan_range(jnp.int32(SWEEP_END), jnp.int32(NUM_BUCKETS),
                   jnp.int32(SWEEP_END))

    def tail_group(_, g16):
        rem = m - g16
        valid = lane < rem
        boffs = plsc.load_gather(wb_v, [lane + g16])
        boffs = jnp.where(valid, boffs, 0)
        wp = plsc.load_gather(wp_v, [lane + g16])
        wp = jnp.where(valid, wp, jnp.int32(0))
        for d in range(HALF):
            dv = jnp.full((L,), d, jnp.int32)
            vals = plsc.load_gather(
                tail_v, [(fbase + dv) * jnp.int32(TAIL) + boffs])
            plsc.store_scatter(stage_v, [dv, wp], vals, mask=valid)
        return g16 + jnp.int32(L)

    lax.fori_loop(0, (m + jnp.int32(L - 1)) // jnp.int32(L), tail_group,
                  jnp.int32(0))

    oof = pl.multiple_of(toff, 2048)
    pltpu.sync_copy(stage_v,
                    out_hbm.at[row, pl.ds(fbase, HALF), pl.ds(oof, CHUNK)])


@jax.jit
def _bigram_embed(ids_flat, tab_t, tail_flat):
    mesh = plsc.VectorSubcoreMesh(core_axis_name="c", subcore_axis_name="s")
    run = functools.partial(
        pl.kernel,
        out_type=jax.ShapeDtypeStruct((BATCH, EMBED_DIM, SEQ_LEN),
                                      jnp.float32),
        mesh=mesh,
        scratch_types=[
            pltpu.VMEM((CHUNK + 16,), jnp.int32),
            pltpu.VMEM((CHUNK,), jnp.int32),
            pltpu.VMEM((CHUNK + 16,), jnp.int32),
            pltpu.VMEM((CHUNK + 16,), jnp.int32),
            pltpu.VMEM((HALF, CHUNK), jnp.float32),
            pltpu.VMEM((8, PW // 8), jnp.float32),
            [pltpu.VMEM((128,), jnp.float32) for _ in range(HALF)],
            pltpu.VMEM((HALF * 128,), jnp.int32),
            pltpu.VMEM((EMBED_DIM * TAIL,), jnp.float32),
            pltpu.VMEM_SHARED((HALF * PW,), jnp.float32),
            pltpu.SemaphoreType.DMA,
            pltpu.SemaphoreType.DMA,
            pltpu.SemaphoreType.DMA,
        ],
        compiler_params=pltpu.CompilerParams(needs_layout_passes=False),
    )(_sc_body)
    return run(ids_flat, tab_t, tail_flat)


def kernel(input_ids, embedding_weight):
    ids_flat = input_ids.reshape(-1).astype(jnp.int32)
    tab_t = embedding_weight.T
    tail_flat = embedding_weight[SWEEP_END:].T.reshape(-1)
    out = _bigram_embed(ids_flat, tab_t, tail_flat)
    return out.transpose(0, 2, 1)


# speedup vs baseline: 2.1061x; 2.1061x over previous
import functools

import jax
import jax.numpy as jnp
from jax import lax
from jax.experimental import pallas as pl
from jax.experimental.pallas import tpu as pltpu
from jax.experimental.pallas import tpu_sc as plsc

NUM_BUCKETS = 1000000
EMBED_DIM = 32
BATCH = 4
SEQ_LEN = 8192
FLAT = BATCH * SEQ_LEN

A_HI = 761000
A_LO = 435761
B_HI = 503000
B_LO = 40503

_INFO = plsc.get_sparse_core_info()
NC = _INFO.num_cores
NS = _INFO.num_subcores
L = _INFO.num_lanes
CHUNK = FLAT // NS
STEPS = CHUNK // L
HALF = EMBED_DIM // NC

PW = 32768
SWEEP_END = 999936
NPIECE = 31
LAST_W = SWEEP_END - (NPIECE - 1) * PW
TAIL = NUM_BUCKETS - SWEEP_END


def _sc_body(ids_hbm, tab_hbm, tail_hbm, out_hbm, ids_v, idx_v, wb_v, wp_v,
             stage_v, chunk_v, cg_refs, gidx_v, tail_v, sh_v, sem, csem,
             rsem):
    c = lax.axis_index("c")
    s = lax.axis_index("s")
    base = s * jnp.int32(CHUNK)
    row = base // jnp.int32(SEQ_LEN)
    toff = base % jnp.int32(SEQ_LEN)
    fbase = c * jnp.int32(HALF)

    zeros = jnp.zeros((L,), jnp.int32)
    lane = lax.iota(jnp.int32, L)

    plsc.store_scatter(ids_v, [lane], zeros)
    pltpu.sync_copy(ids_hbm.at[pl.ds(base, CHUNK)], ids_v.at[pl.ds(8, CHUNK)])

    @pl.when(s % jnp.int32(NS // BATCH) != 0)
    def _():
        pltpu.sync_copy(ids_hbm.at[pl.ds(base - 8, 8)], ids_v.at[pl.ds(0, 8)])

    pltpu.sync_copy(tail_hbm, tail_v)

    a_hi = jnp.int32(A_HI)
    a_lo = jnp.int32(A_LO)
    b_hi = jnp.int32(B_HI)
    b_lo = jnp.int32(B_LO)
    thousand = jnp.int32(1000)
    nbuckets = jnp.int32(NUM_BUCKETS)

    def hash_step(_, off):
        cur = plsc.load_gather(ids_v, [lane + (off + jnp.int32(8))])
        prev = plsc.load_gather(ids_v, [lane + (off + jnp.int32(7))])
        p1 = prev // thousand
        p0 = prev - p1 * thousand
        c1 = cur // thousand
        c0 = cur - c1 * thousand
        h = (a_hi * p1 + a_lo * p0 + b_hi * c1 + b_lo * c0) % nbuckets
        plsc.store_scatter(idx_v, [lane + off], h)
        return off + jnp.int32(L)

    lax.fori_loop(0, STEPS, hash_step, jnp.int32(0))

    def scan_range(plo, phi, ref0):
        vlo = jnp.full((L,), plo, jnp.int32)
        vhi = jnp.full((L,), phi, jnp.int32)
        vref = jnp.full((L,), ref0, jnp.int32)

        def scan_step(_, carry):
            off, cnt = carry
            h = plsc.load_gather(idx_v, [lane + off])
            mask = jnp.logical_and(h >= vlo, h < vhi)
            plsc.store_compressed(wb_v.at[pl.ds(cnt, L)], h - vref, mask=mask)
            plsc.store_compressed(wp_v.at[pl.ds(cnt, L)], lane + off, mask=mask)
            nmask = plsc.all_reduce_population_count(mask)
            cnt = cnt + jnp.max(nmask)
            return off + jnp.int32(L), cnt

        _, m = lax.fori_loop(0, STEPS, scan_step,
                             (jnp.int32(0), jnp.int32(0)))
        return m

    def extract_from_piece(m):
        def group_step(_, g16):
            rem = m - g16
            valid = lane < rem
            boffs = plsc.load_gather(wb_v, [lane + g16])
            boffs = jnp.where(valid, boffs, 0)
            wp = plsc.load_gather(wp_v, [lane + g16])
            wp = jnp.where(valid, wp, jnp.int32(0))
            for d in range(HALF):
                plsc.store_scatter(gidx_v, [lane + jnp.int32(d * L)],
                                   boffs + jnp.int32(d * PW))
            copies = []
            for d in range(HALF):
                copies.append(pltpu.async_copy(
                    sh_v.at[gidx_v.at[pl.ds(d * L, L)]],
                    cg_refs[d],
                    sem))
            for cp in copies:
                cp.wait()
            for d in range(HALF):
                dv = jnp.full((L,), d, jnp.int32)
                vals = cg_refs[d][...]
                plsc.store_scatter(stage_v, [dv, wp], vals, mask=valid)
            return g16 + jnp.int32(L)

        ngroups = (m + jnp.int32(L - 1)) // jnp.int32(L)
        lax.fori_loop(0, ngroups, group_step, jnp.int32(0))

    def chunk_src(pc):
        plo = pc * jnp.int32(PW)
        lo_load = jnp.minimum(plo, jnp.int32(SWEEP_END - PW))
        lo_load = pl.multiple_of(lo_load, 128)
        blk = s % jnp.int32(2)
        part = s // jnp.int32(2)
        col0 = pl.multiple_of(lo_load + part * jnp.int32(PW // 8), 128)
        return tab_hbm.at[pl.ds(fbase + blk * 8, 8), pl.ds(col0, PW // 8)]

    pltpu.async_copy(chunk_src(jnp.int32(0)), chunk_v, csem)

    def piece_body(_, pc):
        plo = pc * jnp.int32(PW)
        phi = jnp.minimum(plo + jnp.int32(PW), jnp.int32(SWEEP_END))
        lo_load = jnp.minimum(plo, jnp.int32(SWEEP_END - PW))
        blk = s % jnp.int32(2)
        part = s // jnp.int32(2)
        pltpu.make_async_copy(chunk_src(pc), chunk_v, csem).wait()
        rows = []
        for d in range(8):
            dglob = blk * 8 + jnp.int32(d)
            rows.append(pltpu.async_copy(
                chunk_v.at[jnp.int32(d)],
                sh_v.at[pl.ds(dglob * jnp.int32(PW)
                              + part * jnp.int32(PW // 8), PW // 8)],
                rsem))
        for r in rows:
            r.wait()

        @pl.when(pc + 1 < jnp.int32(NPIECE))
        def _():
            pltpu.async_copy(chunk_src(pc + 1), chunk_v, csem)

        plsc.subcore_barrier()
        m = scan_range(plo, phi, lo_load)
        extract_from_piece(m)
        plsc.subcore_barrier()
        return pc + jnp.int32(1)

    lax.fori_loop(0, NPIECE, piece_body, jnp.int32(0))

    m = scan_range(jnp.int32(SWEEP_END), jnp.int32(NUM_BUCKETS),
                   jnp.int32(SWEEP_END))

    def tail_group(_, g16):
        rem = m - g16
        valid = lane < rem
        boffs = plsc.load_gather(wb_v, [lane + g16])
        boffs = jnp.where(valid, boffs, 0)
        wp = plsc.load_gather(wp_v, [lane + g16])
        wp = jnp.where(valid, wp, jnp.int32(0))
        for d in range(HALF):
            dv = jnp.full((L,), d, jnp.int32)
            vals = plsc.load_gather(
                tail_v, [(fbase + dv) * jnp.int32(TAIL) + boffs])
            plsc.store_scatter(stage_v, [dv, wp], vals, mask=valid)
        return g16 + jnp.int32(L)

    lax.fori_loop(0, (m + jnp.int32(L - 1)) // jnp.int32(L), tail_group,
                  jnp.int32(0))

    oof = pl.multiple_of(toff, 2048)
    pltpu.sync_copy(stage_v,
                    out_hbm.at[row, pl.ds(fbase, HALF), pl.ds(oof, CHUNK)])


@jax.jit
def _bigram_embed(ids_flat, tab_t, tail_flat):
    mesh = plsc.VectorSubcoreMesh(core_axis_name="c", subcore_axis_name="s")
    run = functools.partial(
        pl.kernel,
        out_type=jax.ShapeDtypeStruct((BATCH, EMBED_DIM, SEQ_LEN),
                                      jnp.float32),
        mesh=mesh,
        scratch_types=[
            pltpu.VMEM((CHUNK + 16,), jnp.int32),
            pltpu.VMEM((CHUNK,), jnp.int32),
            pltpu.VMEM((CHUNK + 16,), jnp.int32),
            pltpu.VMEM((CHUNK + 16,), jnp.int32),
            pltpu.VMEM((HALF, CHUNK), jnp.float32),
            pltpu.VMEM((8, PW // 8), jnp.float32),
            [pltpu.VMEM((L,), jnp.float32) for _ in range(HALF)],
            pltpu.VMEM((HALF * L,), jnp.int32),
            pltpu.VMEM((EMBED_DIM * TAIL,), jnp.float32),
            pltpu.VMEM_SHARED((HALF * PW,), jnp.float32),
            pltpu.SemaphoreType.DMA,
            pltpu.SemaphoreType.DMA,
            pltpu.SemaphoreType.DMA,
        ],
        compiler_params=pltpu.CompilerParams(needs_layout_passes=False),
    )(_sc_body)
    return run(ids_flat, tab_t, tail_flat)


def kernel(input_ids, embedding_weight):
    ids_flat = input_ids.reshape(-1).astype(jnp.int32)
    tab_t = embedding_weight.T
    tail_flat = embedding_weight[SWEEP_END:].T.reshape(-1)
    out = _bigram_embed(ids_flat, tab_t, tail_flat)
    return out.transpose(0, 2, 1)
